# Initial kernel scaffold; baseline (speedup 1.0000x reference)
import jax
import jax.numpy as jnp
from jax.experimental import pallas as pl


def kernel(k_fea, v_fea, q_fea, Wq, Wk, Wv, Wp, temperature, a1, a2, a3, a4):
    raise NotImplementedError("write your pallas kernel here")


# speedup vs baseline: 1.0053x; 1.0053x over previous
import functools
import jax
import jax.numpy as jnp
from jax import lax
from jax.experimental import pallas as pl
from jax.experimental.pallas import tpu as pltpu

_B, _DIM, _H, _W = 2, 384, 224, 224
_HEADS = 8
_C = _DIM // _HEADS
_R = 8
_S = _H // _R
_KS = (_C // 2, _C * 2 // 3, _C * 3 // 4, _C * 4 // 5)


def _split_bf16(x):
    hi = x.astype(jnp.bfloat16)
    lo = (x - hi.astype(jnp.float32)).astype(jnp.bfloat16)
    return hi, lo


def _dot3(a, b, dims):
    ah, al = _split_bf16(a)
    bh, bl = _split_bf16(b)
    d = functools.partial(lax.dot_general, dimension_numbers=(dims, ((), ())),
                          preferred_element_type=jnp.float32)
    return d(ah, bh) + d(ah, bl) + d(al, bh)


def _row_conv(rows, w_ref, i, keep0, keepW):
    y = jnp.zeros((_DIM, _W), jnp.float32)
    for t in range(3):
        v = rows[i + t]
        wl = w_ref[:, t * 3 + 0].reshape(_DIM, 1)
        wc = w_ref[:, t * 3 + 1].reshape(_DIM, 1)
        wr = w_ref[:, t * 3 + 2].reshape(_DIM, 1)
        vl = keep0 * pltpu.roll(v, 1, 1)
        vr = keepW * pltpu.roll(v, _W - 1, 1)
        y = y + wl * vl + wc * v + wr * vr
    return y


def _conv_rows(x_ref, c0_ref, c1_ref, s):
    x = x_ref[0]
    zs = jnp.where(s == _S, 0.0, 1.0)
    rows = [jnp.where(s == 0, 0.0, c0_ref[...]),
            jnp.where(s == 0, 0.0, c1_ref[...])]
    for j in range(_R):
        rows.append(zs * x[:, j, :])
    c0_ref[...] = rows[_R]
    c1_ref[...] = rows[_R + 1]
    return rows


def _qk_norms_body(q_ref, k_ref, wq_ref, wk_ref,
                   denq_out, denk_out,
                   cq0, cq1, ck0, ck1, nq_acc, nk_acc, nq_cmp, nk_cmp):
    s = pl.program_id(1)

    @pl.when(s == 0)
    def _():
        nq_acc[...] = jnp.zeros_like(nq_acc)
        nk_acc[...] = jnp.zeros_like(nk_acc)
        nq_cmp[...] = jnp.zeros_like(nq_cmp)
        nk_cmp[...] = jnp.zeros_like(nk_cmp)

    col = lax.broadcasted_iota(jnp.int32, (1, _W), 1)
    keep0 = jnp.where(col == 0, 0.0, 1.0)
    keepW = jnp.where(col == _W - 1, 0.0, 1.0)

    def side(x_ref, w_ref, c0_ref, c1_ref, n_acc, n_cmp):
        rows = _conv_rows(x_ref, c0_ref, c1_ref, s)
        acc = n_acc[...]
        cmp_ = n_cmp[...]
        for i in range(_R):
            a = s * _R - 1 + i
            va = jnp.where((a >= 0) & (a < _H), 1.0, 0.0)
            y = va * _row_conv(rows, w_ref, i, keep0, keepW)
            val = jnp.sum(y * y, axis=1).reshape(1, _DIM)
            t = acc + val
            cmp_ = cmp_ + jnp.where(jnp.abs(acc) >= jnp.abs(val),
                                    (acc - t) + val, (val - t) + acc)
            acc = t
        n_acc[...] = acc
        n_cmp[...] = cmp_

    side(q_ref, wq_ref, cq0, cq1, nq_acc, nq_cmp)
    side(k_ref, wk_ref, ck0, ck1, nk_acc, nk_cmp)

    @pl.when(s == _S)
    def _():
        denq_out[0] = jnp.maximum(jnp.sqrt(nq_acc[...] + nq_cmp[...]), 1e-12)
        denk_out[0] = jnp.maximum(jnp.sqrt(nk_acc[...] + nk_cmp[...]), 1e-12)


def _qk_norms(q_fea, k_fea, wq9, wk9):
    return pl.pallas_call(
        _qk_norms_body,
        grid=(_B, _S + 1),
        in_specs=[
            pl.BlockSpec((1, _DIM, _R, _W),
                         lambda b, s: (b, 0, jnp.minimum(s, _S - 1), 0)),
            pl.BlockSpec((1, _DIM, _R, _W),
                         lambda b, s: (b, 0, jnp.minimum(s, _S - 1), 0)),
            pl.BlockSpec((_DIM, 9), lambda b, s: (0, 0)),
            pl.BlockSpec((_DIM, 9), lambda b, s: (0, 0)),
        ],
        out_specs=[
            pl.BlockSpec((1, 1, _DIM), lambda b, s: (b, 0, 0)),
            pl.BlockSpec((1, 1, _DIM), lambda b, s: (b, 0, 0)),
        ],
        out_shape=[
            jax.ShapeDtypeStruct((_B, 1, _DIM), jnp.float32),
            jax.ShapeDtypeStruct((_B, 1, _DIM), jnp.float32),
        ],
        scratch_shapes=[
            pltpu.VMEM((_DIM, _W), jnp.float32),
            pltpu.VMEM((_DIM, _W), jnp.float32),
            pltpu.VMEM((_DIM, _W), jnp.float32),
            pltpu.VMEM((_DIM, _W), jnp.float32),
            pltpu.VMEM((1, _DIM), jnp.float32),
            pltpu.VMEM((1, _DIM), jnp.float32),
            pltpu.VMEM((1, _DIM), jnp.float32),
            pltpu.VMEM((1, _DIM), jnp.float32),
        ],
    )(q_fea, k_fea, wq9, wk9)


def _qk_gram_body(q_ref, k_ref, wq_ref, wk_ref, dq_ref, dk_ref,
                  g_out, cq0, cq1, ck0, ck1, g_acc):
    s = pl.program_id(1)

    @pl.when(s == 0)
    def _():
        g_acc[...] = jnp.zeros_like(g_acc)

    col = lax.broadcasted_iota(jnp.int32, (1, _W), 1)
    keep0 = jnp.where(col == 0, 0.0, 1.0)
    keepW = jnp.where(col == _W - 1, 0.0, 1.0)

    def rne_bf16(x):
        u = lax.bitcast_convert_type(x, jnp.uint32)
        u = u + jnp.uint32(0x7FFF) + ((u >> 16) & jnp.uint32(1))
        u = u & jnp.uint32(0xFFFF0000)
        return lax.bitcast_convert_type(u, jnp.float32)

    def side(x_ref, w_ref, d_ref, c0_ref, c1_ref):
        rows = _conv_rows(x_ref, c0_ref, c1_ref, s)
        den = d_ref[0]
        ys = []
        for i in range(_R):
            a = s * _R - 1 + i
            va = jnp.where((a >= 0) & (a < _H), 1.0, 0.0)
            y = va * _row_conv(rows, w_ref, i, keep0, keepW)
            ys.append(rne_bf16(y / den))
        return ys

    yqs = side(q_ref, wq_ref, dq_ref, cq0, cq1)
    yks = side(k_ref, wk_ref, dk_ref, ck0, ck1)

    acc = g_acc[...]
    for i in range(_R):
        acc = acc + lax.dot_general(
            yqs[i], yks[i], (((1,), (1,)), ((), ())),
            preferred_element_type=jnp.float32)
    g_acc[...] = acc

    @pl.when(s == _S)
    def _():
        for h in range(_HEADS):
            g_out[0, h] = g_acc[h * _C:(h + 1) * _C, h * _C:(h + 1) * _C]


def _qk_gram(q_fea, k_fea, wq9, wk9, denq_col, denk_col):
    return pl.pallas_call(
        _qk_gram_body,
        grid=(_B, _S + 1),
        in_specs=[
            pl.BlockSpec((1, _DIM, _R, _W),
                         lambda b, s: (b, 0, jnp.minimum(s, _S - 1), 0)),
            pl.BlockSpec((1, _DIM, _R, _W),
                         lambda b, s: (b, 0, jnp.minimum(s, _S - 1), 0)),
            pl.BlockSpec((_DIM, 9), lambda b, s: (0, 0)),
            pl.BlockSpec((_DIM, 9), lambda b, s: (0, 0)),
            pl.BlockSpec((1, _DIM, 1), lambda b, s: (b, 0, 0)),
            pl.BlockSpec((1, _DIM, 1), lambda b, s: (b, 0, 0)),
        ],
        out_specs=pl.BlockSpec((1, _HEADS, _C, _C), lambda b, s: (b, 0, 0, 0)),
        out_shape=jax.ShapeDtypeStruct((_B, _HEADS, _C, _C), jnp.float32),
        scratch_shapes=[
            pltpu.VMEM((_DIM, _W), jnp.float32),
            pltpu.VMEM((_DIM, _W), jnp.float32),
            pltpu.VMEM((_DIM, _W), jnp.float32),
            pltpu.VMEM((_DIM, _W), jnp.float32),
            pltpu.VMEM((_DIM, _DIM), jnp.float32),
        ],
    )(q_fea, k_fea, wq9, wk9, denq_col, denk_col)


def _combine_body(g_ref, t_ref, c_ref, a_out):
    h = pl.program_id(1)
    g = g_ref[0, 0]
    t = t_ref[h]
    a = g * t
    ai = a.reshape(_C, 1, _C)
    aj = a.reshape(_C, _C, 1)
    gt = ai > aj
    eq = ai == aj
    i_lt_j = (lax.broadcasted_iota(jnp.int32, (1, _C, _C), 2)
              < lax.broadcasted_iota(jnp.int32, (1, _C, _C), 1))
    sel = jnp.where(gt | (eq & i_lt_j), 1.0, 0.0)
    rank = jnp.sum(sel, axis=2)
    e = jnp.exp(a - jnp.max(a, axis=1, keepdims=True))
    scale = jnp.zeros((_C, _C), jnp.float32)
    for idx, kk in enumerate(_KS):
        m = rank < kk
        d = jnp.sum(jnp.where(m, e, 0.0), axis=1, keepdims=True)
        scale = scale + jnp.where(m, c_ref[idx] / d, 0.0)
    a_out[0, 0] = e * scale


def _combine(g, temp, coefs):
    return pl.pallas_call(
        _combine_body,
        grid=(_B, _HEADS),
        in_specs=[
            pl.BlockSpec((1, 1, _C, _C), lambda b, h: (b, h, 0, 0)),
            pl.BlockSpec(memory_space=pltpu.SMEM),
            pl.BlockSpec(memory_space=pltpu.SMEM),
        ],
        out_specs=pl.BlockSpec((1, 1, _C, _C), lambda b, h: (b, h, 0, 0)),
        out_shape=jax.ShapeDtypeStruct((_B, _HEADS, _C, _C), jnp.float32),
    )(g, temp, coefs)


def _apply_body(v_ref, wv_ref, a_ref, wp_ref, out_ref, c_ref, mt_ref):
    s = pl.program_id(1)

    @pl.when(s == 0)
    def _():
        for h in range(_HEADS):
            blk = _dot3(a_ref[0, h], wp_ref[:, h * _C:(h + 1) * _C],
                        ((0,), (1,)))
            mt_ref[h * _C:(h + 1) * _C, :] = blk

    col = lax.broadcasted_iota(jnp.int32, (1, _W), 1)
    keep0 = jnp.where(col == 0, 0.0, 1.0)
    keepW = jnp.where(col == _W - 1, 0.0, 1.0)

    zs = jnp.where(s == _S, 0.0, 1.0)
    x = v_ref[0]
    rows = [jnp.where(s == 1, 0.0, c_ref[0])]
    for j in range(1, _R + 1):
        rows.append(c_ref[j])
    rows.append(zs * x[:, 0, :])
    c_ref[0] = rows[_R]
    for j in range(_R):
        c_ref[j + 1] = zs * x[:, j, :]

    mt = mt_ref[...]
    for i in range(_R):
        y = _row_conv(rows, wv_ref, i, keep0, keepW)
        z = lax.dot_general(mt, y, (((0,), (0,)), ((), ())),
                            preferred_element_type=jnp.float32,
            precision=lax.Precision.HIGHEST)
        out_ref[0, :, i, :] = z


def _apply(v_fea, wv9, a_comb, wp2):
    return pl.pallas_call(
        _apply_body,
        grid=(_B, _S + 1),
        in_specs=[
            pl.BlockSpec((1, _DIM, _R, _W),
                         lambda b, s: (b, 0, jnp.minimum(s, _S - 1), 0)),
            pl.BlockSpec((_DIM, 9), lambda b, s: (0, 0)),
            pl.BlockSpec((1, _HEADS, _C, _C), lambda b, s: (b, 0, 0, 0)),
            pl.BlockSpec((_DIM, _DIM), lambda b, s: (0, 0)),
        ],
        out_specs=pl.BlockSpec(
            (1, _DIM, _R, _W),
            lambda b, s: (b, 0, jnp.maximum(s - 1, 0), 0)),
        out_shape=jax.ShapeDtypeStruct((_B, _DIM, _H, _W), jnp.float32),
        scratch_shapes=[
            pltpu.VMEM((_R + 1, _DIM, _W), jnp.float32),
            pltpu.VMEM((_DIM, _DIM), jnp.float32),
        ],
    )(v_fea, wv9, a_comb, wp2)


@jax.jit
def kernel(k_fea, v_fea, q_fea, Wq, Wk, Wv, Wp, temperature, a1, a2, a3, a4):
    wq9 = Wq.reshape(_DIM, 9)
    wk9 = Wk.reshape(_DIM, 9)
    wv9 = Wv.reshape(_DIM, 9)
    wp2 = Wp.reshape(_DIM, _DIM)
    temp = temperature.reshape(_HEADS)
    coefs = jnp.stack([a1[0], a2[0], a3[0], a4[0]])

    denq, denk = _qk_norms(q_fea, k_fea, wq9, wk9)
    g = _qk_gram(q_fea, k_fea, wq9, wk9,
                 denq.reshape(_B, _DIM, 1), denk.reshape(_B, _DIM, 1))
    a_comb = _combine(g, temp, coefs)
    return _apply(v_fea, wv9, a_comb, wp2)
